# baseline (speedup 1.0000x reference)
import functools
import math

import jax
import jax.numpy as jnp
from jax import lax
from jax.experimental import pallas as pl
from jax.experimental.pallas import tpu as pltpu
from jax.experimental.pallas import tpu_sc as plsc

N_TOK = 2048
D_IN = 1024
D_HID = 2048
D_OUT = 1024
N_EXP = 8
N_PAIR = N_TOK * 2

TM = 256
S = N_PAIR + N_EXP * TM
N_TILES = S // TM

NC, NS = 2, 16
NW = NC * NS
SLOTS_W = S // NW
TOK_W = N_TOK // NW


def _gelu_exact(h):
    return 0.5 * h * (1.0 + lax.erf(h * (1.0 / math.sqrt(2.0))))


def _router_body(x_ref, gw_ref, gb_ref, d0_ref, d1_ref, tk0_ref, tk1_ref,
                 meta_ref):
    x = x_ref[...]
    gw = gw_ref[...]
    gb = gb_ref[...]

    logits = jnp.dot(x, gw, preferred_element_type=jnp.float32) + gb
    m = jnp.max(logits, axis=-1, keepdims=True)
    ex = jnp.exp(logits - m)
    p = ex / jnp.sum(ex, axis=-1, keepdims=True)

    idx = lax.broadcasted_iota(jnp.int32, (N_TOK, N_EXP), 1)
    m1 = jnp.max(p, axis=-1, keepdims=True)
    i1 = jnp.min(jnp.where(p == m1, idx, N_EXP), axis=-1, keepdims=True)
    mask1 = idx == i1
    p2 = jnp.where(mask1, -jnp.inf, p)
    m2 = jnp.max(p2, axis=-1, keepdims=True)
    i2 = jnp.min(jnp.where(p2 == m2, idx, N_EXP), axis=-1, keepdims=True)
    denom = m1 + m2
    w_top1 = m1 / denom
    w_top2 = m2 / denom

    c0 = (idx == i1).astype(jnp.float32)
    c1 = (idx == i2).astype(jnp.float32)
    c = c0 + c1

    rr = lax.broadcasted_iota(jnp.int32, (N_TOK, N_TOK), 0)
    cc = lax.broadcasted_iota(jnp.int32, (N_TOK, N_TOK), 1)
    tri = (cc < rr).astype(jnp.float32)
    cum = jnp.dot(tri, c, preferred_element_type=jnp.float32)

    counts = jnp.sum(c, axis=0, keepdims=True)
    padded = jnp.ceil(counts * (1.0 / TM)) * TM
    r8 = lax.broadcasted_iota(jnp.int32, (N_EXP, N_EXP), 0)
    c8 = lax.broadcasted_iota(jnp.int32, (N_EXP, N_EXP), 1)
    t8 = (r8 < c8).astype(jnp.float32)
    off = jnp.dot(padded, t8, preferred_element_type=jnp.float32)
    total = jnp.sum(padded)

    base = off + cum
    d0_ref[...] = jnp.sum(c0 * base, axis=-1, keepdims=True).astype(jnp.int32)
    d1_ref[...] = jnp.sum(c1 * base, axis=-1, keepdims=True).astype(jnp.int32)
    tk0_ref[...] = w_top1
    tk1_ref[...] = w_top2

    mrow = lax.broadcasted_iota(jnp.int32, (8, 128), 0)
    mcol = lax.broadcasted_iota(jnp.int32, (8, 128), 1)
    tstart = (mcol * TM).astype(jnp.float32)
    lane8 = lax.broadcasted_iota(jnp.int32, (1, N_EXP), 1)
    acc = jnp.zeros((8, 128), jnp.float32)
    for e in range(N_EXP):
        off_e = jnp.sum(jnp.where(lane8 == e, off, 0.0))
        acc = acc + jnp.where(off_e <= tstart, 1.0, 0.0)
    te = acc.astype(jnp.int32) - 1
    valid = (tstart < total).astype(jnp.int32)
    meta_ref[...] = jnp.where(mrow == 1, valid, te)


def _run_router(x, gate_w, gate_b):
    return pl.pallas_call(
        _router_body,
        out_shape=[
            jax.ShapeDtypeStruct((N_TOK, 1), jnp.int32),
            jax.ShapeDtypeStruct((N_TOK, 1), jnp.int32),
            jax.ShapeDtypeStruct((N_TOK, 1), jnp.float32),
            jax.ShapeDtypeStruct((N_TOK, 1), jnp.float32),
            jax.ShapeDtypeStruct((8, 128), jnp.int32),
        ],
    )(x, gate_w, gate_b.reshape(1, N_EXP))


K2_CHUNK = 48
K2_NCH = SLOTS_W // K2_CHUNK


def _route_sc_body(x_hbm, d0_hbm, d1_hbm, tk0_hbm, tk1_hbm, xs_hbm, wts_hbm,
                   d0_v, d1_v, tk0_v, tk1_v, rid_v, wt_v, xbuf0, xbuf1,
                   sem0, sem1):
    wid = lax.axis_index("s") * NC + lax.axis_index("c")
    base = wid * SLOTS_W

    with jax.named_scope("k2_meta_in"):
        pltpu.sync_copy(d0_hbm, d0_v)
        pltpu.sync_copy(d1_hbm, d1_v)
        pltpu.sync_copy(tk0_hbm, tk0_v)
        pltpu.sync_copy(tk1_hbm, tk1_v)

    lane = lax.iota(jnp.int32, 16)
    zf = jnp.zeros((16,), jnp.float32)

    with jax.named_scope("k2_scan"):
        def init_body(i, _):
            rid_v[pl.ds(i * 16, 16)] = (lane + base + i * 16) & (N_TOK - 1)
            wt_v[pl.ds(i * 16, 16)] = zf
            return 0

        lax.fori_loop(0, SLOTS_W // 16, init_body, 0)

        def make_scan(d_ref, tk_ref):
            def scan_body(i, _):
                d = d_ref[pl.ds(i * 16, 16)]
                rel = d - base
                msk = (rel >= 0) & (rel < SLOTS_W)
                relc = jnp.clip(rel, 0, SLOTS_W - 1)
                plsc.store_scatter(rid_v, [relc], lane + i * 16, mask=msk)
                plsc.store_scatter(wt_v, [relc], tk_ref[pl.ds(i * 16, 16)],
                                   mask=msk)
                return 0
            return scan_body

        lax.fori_loop(0, N_TOK // 16, make_scan(d0_v, tk0_v), 0)
        lax.fori_loop(0, N_TOK // 16, make_scan(d1_v, tk1_v), 0)

    with jax.named_scope("k2_gather"):
        bufs = (xbuf0, xbuf1)
        sems = (sem0, sem1)

        def gather(ch):
            b = ch % 2
            return pltpu.async_copy(
                x_hbm.at[rid_v.at[pl.ds(ch * K2_CHUNK, K2_CHUNK)]],
                bufs[b], sems[b])

        cpg = {0: gather(0), 1: gather(1)}
        last_scatter = {}
        for ch in range(K2_NCH):
            b = ch % 2
            cpg[ch].wait()
            cs = pltpu.async_copy(
                bufs[b], xs_hbm.at[pl.ds(base + ch * K2_CHUNK, K2_CHUNK)],
                sems[b])
            if ch + 2 < K2_NCH:
                cs.wait()
                cpg[ch + 2] = gather(ch + 2)
            else:
                last_scatter[b] = cs
        for b in sorted(last_scatter):
            last_scatter[b].wait()
        pltpu.sync_copy(wt_v, wts_hbm.at[pl.ds(base, SLOTS_W)])


def _run_route_sc(x, d0, d1, tk0, tk1):
    f = pl.kernel(
        _route_sc_body,
        out_type=[
            jax.ShapeDtypeStruct((S, D_IN), jnp.float32),
            jax.ShapeDtypeStruct((S,), jnp.float32),
        ],
        mesh=plsc.VectorSubcoreMesh(
            core_axis_name="c", subcore_axis_name="s",
            num_cores=NC, num_subcores=NS),
        scratch_types=[
            pltpu.VMEM((N_TOK,), jnp.int32),
            pltpu.VMEM((N_TOK,), jnp.int32),
            pltpu.VMEM((N_TOK,), jnp.float32),
            pltpu.VMEM((N_TOK,), jnp.float32),
            pltpu.VMEM((SLOTS_W,), jnp.int32),
            pltpu.VMEM((SLOTS_W,), jnp.float32),
            pltpu.VMEM((K2_CHUNK, D_IN), jnp.float32),
            pltpu.VMEM((K2_CHUNK, D_IN), jnp.float32),
            pltpu.SemaphoreType.DMA,
            pltpu.SemaphoreType.DMA,
        ],
        compiler_params=pltpu.CompilerParams(needs_layout_passes=False),
    )
    return f(x, d0, d1, tk0, tk1)


def _expert_body(te_ref, valid_ref, xs_ref, w1_ref, b1_ref, w2_ref, b2_ref,
                 wts_ref, out_ref):
    t = pl.program_id(0)

    @pl.when(valid_ref[t] == 1)
    def _compute():
        xt = xs_ref[...]
        b1 = b1_ref[...].reshape(1, D_HID)
        b2 = b2_ref[...].reshape(1, D_OUT)
        h = jnp.dot(xt, w1_ref[0], preferred_element_type=jnp.float32) + b1
        h = _gelu_exact(h)
        y = jnp.dot(h, w2_ref[0], preferred_element_type=jnp.float32) + b2
        out_ref[...] = y * wts_ref[...]


def _run_experts(te, valid, xs, w1, b1, w2, b2, wts):
    grid_spec = pltpu.PrefetchScalarGridSpec(
        num_scalar_prefetch=2,
        grid=(N_TILES,),
        in_specs=[
            pl.BlockSpec((TM, D_IN), lambda t, te, v: (t, 0)),
            pl.BlockSpec((1, D_IN, D_HID), lambda t, te, v: (te[t], 0, 0)),
            pl.BlockSpec((1, 1, D_HID), lambda t, te, v: (te[t], 0, 0)),
            pl.BlockSpec((1, D_HID, D_OUT), lambda t, te, v: (te[t], 0, 0)),
            pl.BlockSpec((1, 1, D_OUT), lambda t, te, v: (te[t], 0, 0)),
            pl.BlockSpec((TM, 1), lambda t, te, v: (t, 0)),
        ],
        out_specs=pl.BlockSpec((TM, D_OUT), lambda t, te, v: (t, 0)),
    )
    return pl.pallas_call(
        _expert_body,
        grid_spec=grid_spec,
        out_shape=jax.ShapeDtypeStruct((S, D_OUT), jnp.float32),
    )(te, valid, xs, w1, b1.reshape(N_EXP, 1, D_HID),
      w2, b2.reshape(N_EXP, 1, D_OUT), wts)


K4_CHUNK = 16
K4_NCH = TOK_W // K4_CHUNK


def _combine_sc_body(y_hbm, d0_hbm, d1_hbm, out_hbm,
                     d0_v, d1_v, bufa0, bufb0, bufa1, bufb1,
                     gsem0, gsem1, ssem0, ssem1):
    wid = lax.axis_index("s") * NC + lax.axis_index("c")
    tbase = wid * TOK_W

    pltpu.sync_copy(d0_hbm.at[pl.ds(tbase, TOK_W)], d0_v)
    pltpu.sync_copy(d1_hbm.at[pl.ds(tbase, TOK_W)], d1_v)

    bufa = (bufa0, bufa1)
    bufb = (bufb0, bufb1)
    gsem = (gsem0, gsem1)
    ssem = (ssem0, ssem1)

    def gathers(ch):
        s = ch % 2
        sl = pl.ds(ch * K4_CHUNK, K4_CHUNK)
        return (pltpu.async_copy(y_hbm.at[d0_v.at[sl]], bufa[s], gsem[s]),
                pltpu.async_copy(y_hbm.at[d1_v.at[sl]], bufb[s], gsem[s]))

    pend_g = {0: gathers(0)}
    pend_s = {}
    for ch in range(K4_NCH):
        s = ch % 2
        if ch + 1 < K4_NCH:
            if ch - 1 in pend_s:
                pend_s.pop(ch - 1).wait()
            pend_g[ch + 1] = gathers(ch + 1)
        ga, gb = pend_g.pop(ch)
        ga.wait()
        gb.wait()

        def add_row(r, _):
            for seg in range(D_OUT // 16):
                sl = pl.ds(seg * 16, 16)
                bufa[s][r, sl] = bufa[s][r, sl] + bufb[s][r, sl]
            return 0

        lax.fori_loop(0, K4_CHUNK, add_row, 0)
        pend_s[ch] = pltpu.async_copy(
            bufa[s], out_hbm.at[pl.ds(tbase + ch * K4_CHUNK, K4_CHUNK)],
            ssem[s])
    for ch in sorted(pend_s):
        pend_s[ch].wait()


def _run_combine_sc(y, d0, d1):
    f = pl.kernel(
        _combine_sc_body,
        out_type=jax.ShapeDtypeStruct((N_TOK, D_OUT), jnp.float32),
        mesh=plsc.VectorSubcoreMesh(
            core_axis_name="c", subcore_axis_name="s",
            num_cores=NC, num_subcores=NS),
        scratch_types=[
            pltpu.VMEM((TOK_W,), jnp.int32),
            pltpu.VMEM((TOK_W,), jnp.int32),
            pltpu.VMEM((K4_CHUNK, D_OUT), jnp.float32),
            pltpu.VMEM((K4_CHUNK, D_OUT), jnp.float32),
            pltpu.VMEM((K4_CHUNK, D_OUT), jnp.float32),
            pltpu.VMEM((K4_CHUNK, D_OUT), jnp.float32),
            pltpu.SemaphoreType.DMA,
            pltpu.SemaphoreType.DMA,
            pltpu.SemaphoreType.DMA,
            pltpu.SemaphoreType.DMA,
        ],
        compiler_params=pltpu.CompilerParams(needs_layout_passes=False),
    )
    return f(y, d0, d1)


@jax.jit
def kernel(x, gate_w, gate_b, w1, b1, w2, b2):
    d0c, d1c, tk0c, tk1c, meta = _run_router(x, gate_w, gate_b)
    te = meta[0, :N_TILES]
    valid = meta[1, :N_TILES]
    d0 = d0c.reshape(N_TOK)
    d1 = d1c.reshape(N_TOK)

    xs, wts = _run_route_sc(x, d0, d1, tk0c.reshape(N_TOK), tk1c.reshape(N_TOK))
    y = _run_experts(te, valid, xs, w1, b1, w2, b2, wts.reshape(S, 1))
    return _run_combine_sc(y, d0, d1)


# speedup vs baseline: 1.0201x; 1.0201x over previous
import functools
import math

import jax
import jax.numpy as jnp
from jax import lax
from jax.experimental import pallas as pl
from jax.experimental.pallas import tpu as pltpu
from jax.experimental.pallas import tpu_sc as plsc

N_TOK = 2048
D_IN = 1024
D_HID = 2048
D_OUT = 1024
N_EXP = 8
N_PAIR = N_TOK * 2

TM = 256
S = N_PAIR + N_EXP * TM
N_TILES = S // TM

NC, NS = 2, 16
NW = NC * NS
SLOTS_W = S // NW
TOK_W = N_TOK // NW


def _gelu_exact(h):
    return 0.5 * h * (1.0 + lax.erf(h * (1.0 / math.sqrt(2.0))))


def _router_body(x_ref, gw_ref, gb_ref, dest_ref, tkw_ref, meta_ref):
    x = x_ref[...]
    gw = gw_ref[...]
    gb = gb_ref[...]

    logits = jnp.dot(x, gw, preferred_element_type=jnp.float32) + gb
    m = jnp.max(logits, axis=-1, keepdims=True)
    ex = jnp.exp(logits - m)
    p = ex / jnp.sum(ex, axis=-1, keepdims=True)

    idx = lax.broadcasted_iota(jnp.int32, (N_TOK, N_EXP), 1)
    m1 = jnp.max(p, axis=-1, keepdims=True)
    i1 = jnp.min(jnp.where(p == m1, idx, N_EXP), axis=-1, keepdims=True)
    mask1 = idx == i1
    p2 = jnp.where(mask1, -jnp.inf, p)
    m2 = jnp.max(p2, axis=-1, keepdims=True)
    i2 = jnp.min(jnp.where(p2 == m2, idx, N_EXP), axis=-1, keepdims=True)
    denom = m1 + m2
    w_top1 = m1 / denom
    w_top2 = m2 / denom

    c0 = (idx == i1).astype(jnp.float32)
    c1 = (idx == i2).astype(jnp.float32)
    c = c0 + c1

    rr = lax.broadcasted_iota(jnp.int32, (N_TOK, N_TOK), 0)
    cc = lax.broadcasted_iota(jnp.int32, (N_TOK, N_TOK), 1)
    tri = (cc < rr).astype(jnp.float32)
    cum = jnp.dot(tri, c, preferred_element_type=jnp.float32)

    counts = jnp.sum(c, axis=0, keepdims=True)
    padded = jnp.ceil(counts * (1.0 / TM)) * TM
    r8 = lax.broadcasted_iota(jnp.int32, (N_EXP, N_EXP), 0)
    c8 = lax.broadcasted_iota(jnp.int32, (N_EXP, N_EXP), 1)
    t8 = (r8 < c8).astype(jnp.float32)
    off = jnp.dot(padded, t8, preferred_element_type=jnp.float32)
    total = jnp.sum(padded)

    base = off + cum
    dest0 = jnp.sum(c0 * base, axis=-1, keepdims=True)
    dest1 = jnp.sum(c1 * base, axis=-1, keepdims=True)
    even = (idx % 2) == 0
    dest_ref[...] = jnp.where(even, dest0, dest1).astype(jnp.int32)
    tkw_ref[...] = jnp.where(even, w_top1, w_top2)

    mrow = lax.broadcasted_iota(jnp.int32, (8, 128), 0)
    mcol = lax.broadcasted_iota(jnp.int32, (8, 128), 1)
    tstart = (mcol * TM).astype(jnp.float32)
    lane8 = lax.broadcasted_iota(jnp.int32, (1, N_EXP), 1)
    acc = jnp.zeros((8, 128), jnp.float32)
    for e in range(N_EXP):
        off_e = jnp.sum(jnp.where(lane8 == e, off, 0.0))
        acc = acc + jnp.where(off_e <= tstart, 1.0, 0.0)
    te = acc.astype(jnp.int32) - 1
    valid = (tstart < total).astype(jnp.int32)
    meta_ref[...] = jnp.where(mrow == 1, valid, te)


def _run_router(x, gate_w, gate_b):
    return pl.pallas_call(
        _router_body,
        out_shape=[
            jax.ShapeDtypeStruct((N_TOK, N_EXP), jnp.int32),
            jax.ShapeDtypeStruct((N_TOK, N_EXP), jnp.float32),
            jax.ShapeDtypeStruct((8, 128), jnp.int32),
        ],
    )(x, gate_w, gate_b.reshape(1, N_EXP))


K2_CHUNK = 48
K2_NCH = SLOTS_W // K2_CHUNK


def _route_sc_body(x_hbm, d0_hbm, d1_hbm, tk0_hbm, tk1_hbm, xs_hbm, wts_hbm,
                   d0_v, d1_v, tk0_v, tk1_v, rid_v, wt_v, xbuf0, xbuf1,
                   sem0, sem1):
    wid = lax.axis_index("s") * NC + lax.axis_index("c")
    base = wid * SLOTS_W

    with jax.named_scope("k2_meta_in"):
        pltpu.sync_copy(d0_hbm, d0_v)
        pltpu.sync_copy(d1_hbm, d1_v)
        pltpu.sync_copy(tk0_hbm, tk0_v)
        pltpu.sync_copy(tk1_hbm, tk1_v)

    lane = lax.iota(jnp.int32, 16)
    zf = jnp.zeros((16,), jnp.float32)

    with jax.named_scope("k2_scan"):
        def init_body(i, _):
            rid_v[pl.ds(i * 16, 16)] = (lane + base + i * 16) & (N_TOK - 1)
            wt_v[pl.ds(i * 16, 16)] = zf
            return 0

        lax.fori_loop(0, SLOTS_W // 16, init_body, 0)

        def make_scan(d_ref, tk_ref):
            def scan_body(i, _):
                d = d_ref[pl.ds(i * 16, 16)]
                rel = d - base
                msk = (rel >= 0) & (rel < SLOTS_W)
                relc = jnp.clip(rel, 0, SLOTS_W - 1)
                plsc.store_scatter(rid_v, [relc], lane + i * 16, mask=msk)
                plsc.store_scatter(wt_v, [relc], tk_ref[pl.ds(i * 16, 16)],
                                   mask=msk)
                return 0
            return scan_body

        lax.fori_loop(0, N_TOK // 16, make_scan(d0_v, tk0_v), 0)
        lax.fori_loop(0, N_TOK // 16, make_scan(d1_v, tk1_v), 0)

    with jax.named_scope("k2_gather"):
        bufs = (xbuf0, xbuf1)
        sems = (sem0, sem1)

        def gather(ch):
            b = ch % 2
            return pltpu.async_copy(
                x_hbm.at[rid_v.at[pl.ds(ch * K2_CHUNK, K2_CHUNK)]],
                bufs[b], sems[b])

        cpg = {0: gather(0), 1: gather(1)}
        last_scatter = {}
        for ch in range(K2_NCH):
            b = ch % 2
            cpg[ch].wait()
            cs = pltpu.async_copy(
                bufs[b], xs_hbm.at[pl.ds(base + ch * K2_CHUNK, K2_CHUNK)],
                sems[b])
            if ch + 2 < K2_NCH:
                cs.wait()
                cpg[ch + 2] = gather(ch + 2)
            else:
                last_scatter[b] = cs
        for b in sorted(last_scatter):
            last_scatter[b].wait()
        pltpu.sync_copy(wt_v, wts_hbm.at[pl.ds(base, SLOTS_W)])


def _run_route_sc(x, d0, d1, tk0, tk1):
    f = pl.kernel(
        _route_sc_body,
        out_type=[
            jax.ShapeDtypeStruct((S, D_IN), jnp.float32),
            jax.ShapeDtypeStruct((S,), jnp.float32),
        ],
        mesh=plsc.VectorSubcoreMesh(
            core_axis_name="c", subcore_axis_name="s",
            num_cores=NC, num_subcores=NS),
        scratch_types=[
            pltpu.VMEM((N_TOK,), jnp.int32),
            pltpu.VMEM((N_TOK,), jnp.int32),
            pltpu.VMEM((N_TOK,), jnp.float32),
            pltpu.VMEM((N_TOK,), jnp.float32),
            pltpu.VMEM((SLOTS_W,), jnp.int32),
            pltpu.VMEM((SLOTS_W,), jnp.float32),
            pltpu.VMEM((K2_CHUNK, D_IN), jnp.float32),
            pltpu.VMEM((K2_CHUNK, D_IN), jnp.float32),
            pltpu.SemaphoreType.DMA,
            pltpu.SemaphoreType.DMA,
        ],
        compiler_params=pltpu.CompilerParams(needs_layout_passes=False),
    )
    return f(x, d0, d1, tk0, tk1)


def _expert_body(te_ref, valid_ref, xs_ref, w1_ref, b1_ref, w2_ref, b2_ref,
                 wts_ref, out_ref):
    t = pl.program_id(0)

    @pl.when(valid_ref[t] == 1)
    def _compute():
        xt = xs_ref[...]
        b1 = b1_ref[...].reshape(1, D_HID)
        b2 = b2_ref[...].reshape(1, D_OUT)
        h = jnp.dot(xt, w1_ref[0], preferred_element_type=jnp.float32) + b1
        h = _gelu_exact(h)
        y = jnp.dot(h, w2_ref[0], preferred_element_type=jnp.float32) + b2
        out_ref[...] = y * wts_ref[...]


def _run_experts(te, valid, xs, w1, b1, w2, b2, wts):
    grid_spec = pltpu.PrefetchScalarGridSpec(
        num_scalar_prefetch=2,
        grid=(N_TILES,),
        in_specs=[
            pl.BlockSpec((TM, D_IN), lambda t, te, v: (t, 0)),
            pl.BlockSpec((1, D_IN, D_HID), lambda t, te, v: (te[t], 0, 0)),
            pl.BlockSpec((1, 1, D_HID), lambda t, te, v: (te[t], 0, 0)),
            pl.BlockSpec((1, D_HID, D_OUT), lambda t, te, v: (te[t], 0, 0)),
            pl.BlockSpec((1, 1, D_OUT), lambda t, te, v: (te[t], 0, 0)),
            pl.BlockSpec((TM, 1), lambda t, te, v: (t, 0)),
        ],
        out_specs=pl.BlockSpec((TM, D_OUT), lambda t, te, v: (t, 0)),
    )
    return pl.pallas_call(
        _expert_body,
        grid_spec=grid_spec,
        out_shape=jax.ShapeDtypeStruct((S, D_OUT), jnp.float32),
    )(te, valid, xs, w1, b1.reshape(N_EXP, 1, D_HID),
      w2, b2.reshape(N_EXP, 1, D_OUT), wts)


K4_CHUNK = 16
K4_NCH = TOK_W // K4_CHUNK


def _combine_sc_body(y_hbm, d0_hbm, d1_hbm, out_hbm,
                     d0_v, d1_v, bufa0, bufb0, bufa1, bufb1,
                     gsem0, gsem1, ssem0, ssem1):
    wid = lax.axis_index("s") * NC + lax.axis_index("c")
    tbase = wid * TOK_W

    pltpu.sync_copy(d0_hbm.at[pl.ds(tbase, TOK_W)], d0_v)
    pltpu.sync_copy(d1_hbm.at[pl.ds(tbase, TOK_W)], d1_v)

    bufa = (bufa0, bufa1)
    bufb = (bufb0, bufb1)
    gsem = (gsem0, gsem1)
    ssem = (ssem0, ssem1)

    def gathers(ch):
        s = ch % 2
        sl = pl.ds(ch * K4_CHUNK, K4_CHUNK)
        return (pltpu.async_copy(y_hbm.at[d0_v.at[sl]], bufa[s], gsem[s]),
                pltpu.async_copy(y_hbm.at[d1_v.at[sl]], bufb[s], gsem[s]))

    pend_g = {0: gathers(0)}
    pend_s = {}
    for ch in range(K4_NCH):
        s = ch % 2
        if ch + 1 < K4_NCH:
            if ch - 1 in pend_s:
                pend_s.pop(ch - 1).wait()
            pend_g[ch + 1] = gathers(ch + 1)
        ga, gb = pend_g.pop(ch)
        ga.wait()
        gb.wait()

        def add_row(r, _):
            for seg in range(D_OUT // 16):
                sl = pl.ds(seg * 16, 16)
                bufa[s][r, sl] = bufa[s][r, sl] + bufb[s][r, sl]
            return 0

        lax.fori_loop(0, K4_CHUNK, add_row, 0)
        pend_s[ch] = pltpu.async_copy(
            bufa[s], out_hbm.at[pl.ds(tbase + ch * K4_CHUNK, K4_CHUNK)],
            ssem[s])
    for ch in sorted(pend_s):
        pend_s[ch].wait()


def _run_combine_sc(y, d0, d1):
    f = pl.kernel(
        _combine_sc_body,
        out_type=jax.ShapeDtypeStruct((N_TOK, D_OUT), jnp.float32),
        mesh=plsc.VectorSubcoreMesh(
            core_axis_name="c", subcore_axis_name="s",
            num_cores=NC, num_subcores=NS),
        scratch_types=[
            pltpu.VMEM((TOK_W,), jnp.int32),
            pltpu.VMEM((TOK_W,), jnp.int32),
            pltpu.VMEM((K4_CHUNK, D_OUT), jnp.float32),
            pltpu.VMEM((K4_CHUNK, D_OUT), jnp.float32),
            pltpu.VMEM((K4_CHUNK, D_OUT), jnp.float32),
            pltpu.VMEM((K4_CHUNK, D_OUT), jnp.float32),
            pltpu.SemaphoreType.DMA,
            pltpu.SemaphoreType.DMA,
            pltpu.SemaphoreType.DMA,
            pltpu.SemaphoreType.DMA,
        ],
        compiler_params=pltpu.CompilerParams(needs_layout_passes=False),
    )
    return f(y, d0, d1)


@jax.jit
def kernel(x, gate_w, gate_b, w1, b1, w2, b2):
    dest8, tkw8, meta = _run_router(x, gate_w, gate_b)
    te = meta[0, :N_TILES]
    valid = meta[1, :N_TILES]
    d0 = dest8[:, 0]
    d1 = dest8[:, 1]

    xs, wts = _run_route_sc(x, d0, d1, tkw8[:, 0], tkw8[:, 1])
    y = _run_experts(te, valid, xs, w1, b1, w2, b2, wts.reshape(S, 1))
    return _run_combine_sc(y, d0, d1)
